# Initial kernel scaffold; baseline (speedup 1.0000x reference)
import jax
import jax.numpy as jnp
from jax.experimental import pallas as pl


def kernel(x, batch, W_emb1, b_emb1, W_emb2, b_emb2, W_prop, b_prop):
    raise NotImplementedError("write your pallas kernel here")


# speedup vs baseline: 1.8721x; 1.8721x over previous
import jax
import jax.numpy as jnp
from jax.experimental import pallas as pl

N = 100000
D = 128
G = 256
TILE = 1024


def _fused_kernel(x_ref, b_ref, w1_ref, b1_ref, w2_ref, b2_ref, wp_ref,
                  bp_ref, out_ref):
    i = pl.program_id(0)

    @pl.when(i == 0)
    def _init():
        out_ref[...] = jnp.zeros_like(out_ref)

    x = x_ref[...]
    h = jnp.maximum(jnp.dot(x, w1_ref[...], preferred_element_type=jnp.float32)
                    + b1_ref[...], 0.0)
    h = jnp.maximum(jnp.dot(h, w2_ref[...], preferred_element_type=jnp.float32)
                    + b2_ref[...], 0.0)
    p = jnp.dot(h, wp_ref[...], preferred_element_type=jnp.float32) \
        + bp_ref[...]

    row = i * TILE + jax.lax.broadcasted_iota(jnp.int32, (TILE, 1), 0)
    p = jnp.where(row < N, p, 0.0)

    b = b_ref[0, 0, :].reshape(TILE, 1)
    seg = jax.lax.broadcasted_iota(jnp.int32, (TILE, G), 1)
    onehot = jnp.where(b == seg, 1.0, 0.0)
    contrib = jax.lax.dot_general(
        p, onehot, (((0,), (0,)), ((), ())),
        preferred_element_type=jnp.float32)
    out_ref[...] += contrib


def kernel(x, batch, W_emb1, b_emb1, W_emb2, b_emb2, W_prop, b_prop):
    n_tiles = pl.cdiv(N, TILE)
    n_pad = n_tiles * TILE
    batch_p = jnp.pad(batch.astype(jnp.int32), (0, n_pad - N),
                      constant_values=G).reshape(n_tiles, 1, TILE)

    out = pl.pallas_call(
        _fused_kernel,
        grid=(n_tiles,),
        in_specs=[
            pl.BlockSpec((TILE, D), lambda i: (i, 0)),
            pl.BlockSpec((1, 1, TILE), lambda i: (i, 0, 0)),
            pl.BlockSpec((D, D), lambda i: (0, 0)),
            pl.BlockSpec((1, D), lambda i: (0, 0)),
            pl.BlockSpec((D, D), lambda i: (0, 0)),
            pl.BlockSpec((1, D), lambda i: (0, 0)),
            pl.BlockSpec((D, 1), lambda i: (0, 0)),
            pl.BlockSpec((1, 1), lambda i: (0, 0)),
        ],
        out_specs=pl.BlockSpec((1, G), lambda i: (0, 0)),
        out_shape=jax.ShapeDtypeStruct((1, G), jnp.float32),
    )(x, batch_p, W_emb1, b_emb1.reshape(1, D), W_emb2,
      b_emb2.reshape(1, D), W_prop, b_prop.reshape(1, 1))
    return out[0]


# speedup vs baseline: 2.2513x; 1.2025x over previous
import functools

import jax
import jax.numpy as jnp
from jax import lax
from jax.experimental import pallas as pl
from jax.experimental.pallas import tpu as pltpu
from jax.experimental.pallas import tpu_sc as plsc

N = 100000
D = 128
G = 256
TILE = 2048
N_TILES = 49
N_PAD = N_TILES * TILE

NS = 16
L = 16
CHUNK = N_PAD // NS
VECS = CHUNK // L


def _mlp_kernel(x_ref, w1_ref, b1_ref, w2_ref, b2_ref, wp_ref, bp_ref,
                out_ref):
    i = pl.program_id(0)
    x = x_ref[...]
    h = jnp.maximum(jnp.dot(x, w1_ref[...], preferred_element_type=jnp.float32)
                    + b1_ref[...], 0.0)
    h = jnp.maximum(jnp.dot(h, w2_ref[...], preferred_element_type=jnp.float32)
                    + b2_ref[...], 0.0)
    p = lax.dot_general(wp_ref[...], h, (((0,), (1,)), ((), ())),
                        preferred_element_type=jnp.float32) + bp_ref[...]
    col = i * TILE + lax.broadcasted_iota(jnp.int32, (1, TILE), 1)
    p = jnp.where(col < N, p, 0.0)
    out_ref[...] = p.reshape(TILE // 128, 128)


def _node_property(x, W_emb1, b_emb1, W_emb2, b_emb2, W_prop, b_prop):
    out = pl.pallas_call(
        _mlp_kernel,
        grid=(N_TILES,),
        in_specs=[
            pl.BlockSpec((TILE, D), lambda i: (i, 0)),
            pl.BlockSpec((D, D), lambda i: (0, 0)),
            pl.BlockSpec((1, D), lambda i: (0, 0)),
            pl.BlockSpec((D, D), lambda i: (0, 0)),
            pl.BlockSpec((1, D), lambda i: (0, 0)),
            pl.BlockSpec((D, 1), lambda i: (0, 0)),
            pl.BlockSpec((1, 1), lambda i: (0, 0)),
        ],
        out_specs=pl.BlockSpec((TILE // 128, 128), lambda i: (i, 0)),
        out_shape=jax.ShapeDtypeStruct((N_PAD // 128, 128), jnp.float32),
    )(x, W_emb1, b_emb1.reshape(1, D), W_emb2, b_emb2.reshape(1, D),
      W_prop, b_prop.reshape(1, 1))
    return out.reshape(N_PAD)


_mesh = plsc.VectorSubcoreMesh(core_axis_name="c", subcore_axis_name="s",
                               num_cores=1, num_subcores=NS)


@functools.partial(
    pl.kernel,
    out_type=jax.ShapeDtypeStruct((G,), jnp.float32),
    mesh=_mesh,
    compiler_params=pltpu.CompilerParams(needs_layout_passes=False),
    scratch_types=[
        pltpu.VMEM((CHUNK,), jnp.int32),
        pltpu.VMEM((CHUNK,), jnp.float32),
        pltpu.VMEM((L * G,), jnp.float32),
        pltpu.VMEM((G,), jnp.float32),
        pltpu.VMEM_SHARED((NS, G), jnp.float32),
        pltpu.VMEM((NS, G), jnp.float32),
    ],
)
def _segment_sum(p_hbm, batch_hbm, out_hbm, idx_v, p_v, acc_v, tot_v,
                 shared, all_v):
    sid = lax.axis_index("s")
    base = sid * CHUNK
    pltpu.sync_copy(batch_hbm.at[pl.ds(base, CHUNK)], idx_v)
    pltpu.sync_copy(p_hbm.at[pl.ds(base, CHUNK)], p_v)

    zeros = jnp.zeros((L,), jnp.float32)
    lane_base = lax.broadcasted_iota(jnp.int32, (L,), 0) * G

    def _zero(j, _):
        acc_v[pl.ds(pl.multiple_of(j * L, L), L)] = zeros
        return 0

    lax.fori_loop(0, L * G // L, _zero, 0)

    def _scatter(i, _):
        s = pl.multiple_of(i * L, L)
        idx = idx_v[pl.ds(s, L)]
        vals = p_v[pl.ds(s, L)]
        plsc.addupdate_scatter(acc_v, [lane_base + idx], vals)
        return 0

    lax.fori_loop(0, VECS, _scatter, 0)

    for j in range(G // L):
        v = zeros
        for l in range(L):
            v = v + acc_v[pl.ds(l * G + j * L, L)]
        tot_v[pl.ds(j * L, L)] = v

    pltpu.sync_copy(tot_v, shared.at[sid])
    plsc.subcore_barrier()

    @pl.when(sid == 0)
    def _():
        pltpu.sync_copy(shared, all_v)
        for j in range(G // L):
            v = zeros
            for r in range(NS):
                v = v + all_v[r, pl.ds(j * L, L)]
            tot_v[pl.ds(j * L, L)] = v
        pltpu.sync_copy(tot_v, out_hbm)


def kernel(x, batch, W_emb1, b_emb1, W_emb2, b_emb2, W_prop, b_prop):
    p = _node_property(x, W_emb1, b_emb1, W_emb2, b_emb2, W_prop, b_prop)
    batch_p = jnp.pad(batch.astype(jnp.int32), (0, N_PAD - N),
                      constant_values=G - 1)
    return _segment_sum(p, batch_p)
